# initial kernel scaffold (unmeasured)
import jax
import jax.numpy as jnp
from jax import lax
from jax.experimental import pallas as pl
from jax.experimental.pallas import tpu as pltpu

N_DEV = 32


def kernel(x, w_mat):
    m_per, k = x.shape
    _, n = w_mat.shape
    n_per = n // N_DEV

    def body(x_ref, w_ref, out_ref, y_ref, send_sem, recv_sem, local_sem):
        my = lax.axis_index("i")

        y = jnp.dot(x_ref[:, :], w_ref[:, :], preferred_element_type=jnp.float32)
        y_ref[:, :] = y * jax.nn.sigmoid(y)

        dst = out_ref.at[pl.ds(my * m_per, m_per), :]
        for t in range(N_DEV):
            src = y_ref.at[:, t * n_per:(t + 1) * n_per]

            @pl.when(t == my)
            def _():
                pltpu.make_async_copy(src, dst, local_sem).start()

            @pl.when(t != my)
            def _():
                pltpu.make_async_remote_copy(
                    src_ref=src,
                    dst_ref=dst,
                    send_sem=send_sem,
                    recv_sem=recv_sem,
                    device_id=(t,),
                    device_id_type=pl.DeviceIdType.MESH,
                ).start()

        pltpu.make_async_copy(
            y_ref.at[:, 0:n_per], out_ref.at[pl.ds(0, m_per), :], local_sem
        ).wait()
        for _ in range(N_DEV - 1):
            dummy = pltpu.make_async_remote_copy(
                src_ref=y_ref.at[:, 0:n_per],
                dst_ref=out_ref.at[pl.ds(0, m_per), :],
                send_sem=send_sem,
                recv_sem=recv_sem,
                device_id=(0,),
                device_id_type=pl.DeviceIdType.MESH,
            )
            dummy.wait_send()
            dummy.wait_recv()

    return pl.pallas_call(
        body,
        out_shape=jax.ShapeDtypeStruct((N_DEV * m_per, n_per), jnp.float32),
        in_specs=[
            pl.BlockSpec(memory_space=pltpu.VMEM),
            pl.BlockSpec(memory_space=pltpu.VMEM),
        ],
        out_specs=pl.BlockSpec(memory_space=pltpu.VMEM),
        scratch_shapes=[
            pltpu.VMEM((m_per, n), jnp.float32),
            pltpu.SemaphoreType.DMA,
            pltpu.SemaphoreType.DMA,
            pltpu.SemaphoreType.DMA,
        ],
    )(x, w_mat)


# baseline (device time: 56872 ns/iter reference)
import jax
import jax.numpy as jnp
from jax import lax
from jax.experimental import pallas as pl
from jax.experimental.pallas import tpu as pltpu

N_DEV = 32


def kernel(x, w_mat):
    m_per, k = x.shape
    _, n = w_mat.shape
    n_per = n // N_DEV

    def body(x_ref, w_ref, out_ref, y_ref, send_sem, recv_sem, local_sem):
        my = lax.axis_index("i")

        y = jnp.dot(x_ref[:, :], w_ref[:, :], preferred_element_type=jnp.float32)
        y = y * jax.nn.sigmoid(y)
        for t in range(N_DEV):
            y_ref[t] = y[:, t * n_per:(t + 1) * n_per]

        dst = out_ref.at[pl.ds(my * m_per, m_per), :]
        pltpu.make_async_copy(y_ref.at[my], dst, local_sem).start()
        for t in range(N_DEV):
            @pl.when(t != my)
            def _():
                pltpu.make_async_remote_copy(
                    src_ref=y_ref.at[t],
                    dst_ref=dst,
                    send_sem=send_sem,
                    recv_sem=recv_sem,
                    device_id=(t,),
                    device_id_type=pl.DeviceIdType.MESH,
                ).start()

        pltpu.make_async_copy(
            y_ref.at[0], out_ref.at[pl.ds(0, m_per), :], local_sem
        ).wait()
        for _ in range(N_DEV - 1):
            dummy = pltpu.make_async_remote_copy(
                src_ref=y_ref.at[0],
                dst_ref=out_ref.at[pl.ds(0, m_per), :],
                send_sem=send_sem,
                recv_sem=recv_sem,
                device_id=(0,),
                device_id_type=pl.DeviceIdType.MESH,
            )
            dummy.wait_send()
            dummy.wait_recv()

    return pl.pallas_call(
        body,
        out_shape=jax.ShapeDtypeStruct((N_DEV * m_per, n_per), jnp.float32),
        in_specs=[
            pl.BlockSpec(memory_space=pltpu.VMEM),
            pl.BlockSpec(memory_space=pltpu.VMEM),
        ],
        out_specs=pl.BlockSpec(memory_space=pltpu.VMEM),
        scratch_shapes=[
            pltpu.VMEM((N_DEV, m_per, n_per), jnp.float32),
            pltpu.SemaphoreType.DMA,
            pltpu.SemaphoreType.DMA,
            pltpu.SemaphoreType.DMA,
        ],
        compiler_params=pltpu.CompilerParams(
            vmem_limit_bytes=100 * 1024 * 1024,
        ),
    )(x, w_mat)


# device time: 54782 ns/iter; 1.0382x vs baseline; 1.0382x over previous
import jax
import jax.numpy as jnp
from jax import lax
from jax.experimental import pallas as pl
from jax.experimental.pallas import tpu as pltpu

N_DEV = 32


def kernel(x, w_mat):
    m_per, k = x.shape
    _, n = w_mat.shape
    n_per = n // N_DEV

    def body(x_ref, w_ref, out_ref, y_ref, send_sems, recv_sems, local_sem):
        my = lax.axis_index("i")

        y = jnp.dot(
            x_ref[:, :].astype(jnp.bfloat16),
            w_ref[:, :].astype(jnp.bfloat16),
            preferred_element_type=jnp.float32,
        )
        y = y * jax.nn.sigmoid(y)
        for t in range(N_DEV):
            y_ref[t] = y[:, t * n_per:(t + 1) * n_per]

        dst = out_ref.at[pl.ds(my * m_per, m_per), :]
        pltpu.make_async_copy(y_ref.at[my], dst, local_sem).start()
        for h in range(1, N_DEV):
            t = lax.rem(my + h, N_DEV)
            pltpu.make_async_remote_copy(
                src_ref=y_ref.at[t],
                dst_ref=dst,
                send_sem=send_sems.at[h - 1],
                recv_sem=recv_sems.at[h - 1],
                device_id=(t,),
                device_id_type=pl.DeviceIdType.MESH,
            ).start()

        pltpu.make_async_copy(
            y_ref.at[0], out_ref.at[pl.ds(0, m_per), :], local_sem
        ).wait()
        for h in range(1, N_DEV):
            dummy = pltpu.make_async_remote_copy(
                src_ref=y_ref.at[0],
                dst_ref=out_ref.at[pl.ds(0, m_per), :],
                send_sem=send_sems.at[h - 1],
                recv_sem=recv_sems.at[h - 1],
                device_id=(0,),
                device_id_type=pl.DeviceIdType.MESH,
            )
            dummy.wait_send()
            dummy.wait_recv()

    return pl.pallas_call(
        body,
        out_shape=jax.ShapeDtypeStruct((N_DEV * m_per, n_per), jnp.float32),
        in_specs=[
            pl.BlockSpec(memory_space=pltpu.VMEM),
            pl.BlockSpec(memory_space=pltpu.VMEM),
        ],
        out_specs=pl.BlockSpec(memory_space=pltpu.VMEM),
        scratch_shapes=[
            pltpu.VMEM((N_DEV, m_per, n_per), jnp.float32),
            pltpu.SemaphoreType.DMA((N_DEV - 1,)),
            pltpu.SemaphoreType.DMA((N_DEV - 1,)),
            pltpu.SemaphoreType.DMA,
        ],
        compiler_params=pltpu.CompilerParams(
            vmem_limit_bytes=100 * 1024 * 1024,
        ),
    )(x, w_mat)
